# baseline (speedup 1.0000x reference)
import functools

import jax
import jax.numpy as jnp
from jax import lax
from jax.experimental import pallas as pl
from jax.experimental.pallas import tpu as pltpu
from jax.experimental.pallas import tpu_sc as plsc

C, H, W = 3, 4096, 4096
N_SITES = 1228
PAD_SITES = 1280
BH = 512


def _sc_mask_body(sites_hbm, mask_hbm, idx_v, mask_v):
    cid = lax.axis_index("c")
    sid = lax.axis_index("s")

    @pl.when((cid == 0) & (sid == 0))
    def _():
        pltpu.sync_copy(sites_hbm, idx_v)
        zeros16 = jnp.zeros((16,), jnp.float32)

        def zero_body(i, carry):
            mask_v[pl.ds(i * 16, 16)] = zeros16
            return carry

        lax.fori_loop(0, W // 16, zero_body, 0)
        ones16 = jnp.ones((16,), jnp.float32)

        def scatter_body(i, carry):
            idx = idx_v[pl.ds(i * 16, 16)]
            plsc.store_scatter(mask_v, [idx], ones16, mask=idx < W)
            return carry

        lax.fori_loop(0, PAD_SITES // 16, scatter_body, 0)
        pltpu.sync_copy(mask_v, mask_hbm)


_sc_mask = functools.partial(
    pl.kernel,
    out_type=jax.ShapeDtypeStruct((W,), jnp.float32),
    mesh=plsc.VectorSubcoreMesh(core_axis_name="c", subcore_axis_name="s"),
    scratch_types=[
        pltpu.VMEM((PAD_SITES,), jnp.int32),
        pltpu.VMEM((W,), jnp.float32),
    ],
    compiler_params=pltpu.CompilerParams(needs_layout_passes=False),
)(_sc_mask_body)


def _apply_kernel(mask_ref, x_ref, out_ref):
    c = pl.program_id(0)
    xb = x_ref[...]

    @pl.when(c < C - 1)
    def _copy():
        out_ref[...] = xb

    @pl.when(c == C - 1)
    def _flip():
        m = mask_ref[...].reshape(1, 1, W)
        out_ref[...] = jnp.where(m > 0.0, 1.0 - xb, xb)


@jax.jit
def kernel(x, mask_sites):
    sites = mask_sites.astype(jnp.int32)
    sites = jnp.pad(sites, (0, PAD_SITES - N_SITES), constant_values=jnp.int32(1 << 30))

    mask = _sc_mask(sites).reshape(1, W)

    out = pl.pallas_call(
        _apply_kernel,
        grid=(C, H // BH),
        in_specs=[
            pl.BlockSpec((1, W), lambda c, h: (0, 0)),
            pl.BlockSpec((1, BH, W), lambda c, h: ((c + 1) % C, h, 0)),
        ],
        out_specs=pl.BlockSpec((1, BH, W), lambda c, h: ((c + 1) % C, h, 0)),
        out_shape=jax.ShapeDtypeStruct((C, H, W), jnp.float32),
    )(mask, x)
    return out


# speedup vs baseline: 1.0396x; 1.0396x over previous
import functools

import jax
import jax.numpy as jnp
from jax import lax
from jax.experimental import pallas as pl
from jax.experimental.pallas import tpu as pltpu
from jax.experimental.pallas import tpu_sc as plsc

C, H, W = 3, 4096, 4096
N_SITES = 1228
PAD_SITES = 1280
BH = 512


def _sc_mask_body(sites_hbm, mask_hbm, idx_v, mask_v):
    cid = lax.axis_index("c")
    sid = lax.axis_index("s")

    @pl.when((cid == 0) & (sid == 0))
    def _():
        pltpu.sync_copy(sites_hbm, idx_v)
        zeros16 = jnp.zeros((16,), jnp.float32)

        def zero_body(i, carry):
            mask_v[pl.ds(i * 16, 16)] = zeros16
            return carry

        lax.fori_loop(0, W // 16, zero_body, 0)
        ones16 = jnp.ones((16,), jnp.float32)

        def scatter_body(i, carry):
            idx = idx_v[pl.ds(i * 16, 16)]
            plsc.store_scatter(mask_v, [idx], ones16, mask=idx < W)
            return carry

        lax.fori_loop(0, PAD_SITES // 16, scatter_body, 0)
        pltpu.sync_copy(mask_v, mask_hbm)


_sc_mask = functools.partial(
    pl.kernel,
    out_type=jax.ShapeDtypeStruct((W,), jnp.float32),
    mesh=plsc.VectorSubcoreMesh(core_axis_name="c", subcore_axis_name="s"),
    scratch_types=[
        pltpu.VMEM((PAD_SITES,), jnp.int32),
        pltpu.VMEM((W,), jnp.float32),
    ],
    compiler_params=pltpu.CompilerParams(needs_layout_passes=False),
)(_sc_mask_body)


def _copy12_kernel(x_ref, out_ref):
    out_ref[...] = x_ref[...]


def _flip0_kernel(prev_ref, mask_ref, x_ref, out_ref):
    del prev_ref
    xb = x_ref[...]
    m = mask_ref[...].reshape(1, 1, W)
    out_ref[...] = jnp.where(m > 0.0, 1.0 - xb, xb)


@jax.jit
def kernel(x, mask_sites):
    sites = mask_sites.astype(jnp.int32)
    sites = jnp.pad(sites, (0, PAD_SITES - N_SITES), constant_values=jnp.int32(1 << 30))

    mask = _sc_mask(sites).reshape(1, W)

    partial = pl.pallas_call(
        _copy12_kernel,
        grid=(C - 1, H // BH),
        in_specs=[pl.BlockSpec((1, BH, W), lambda c, h: (c + 1, h, 0))],
        out_specs=pl.BlockSpec((1, BH, W), lambda c, h: (c + 1, h, 0)),
        out_shape=jax.ShapeDtypeStruct((C, H, W), jnp.float32),
    )(x)

    out = pl.pallas_call(
        _flip0_kernel,
        grid=(H // BH,),
        in_specs=[
            pl.BlockSpec(memory_space=pl.ANY),
            pl.BlockSpec((1, W), lambda h: (0, 0)),
            pl.BlockSpec((1, BH, W), lambda h: (0, h, 0)),
        ],
        out_specs=pl.BlockSpec((1, BH, W), lambda h: (0, h, 0)),
        out_shape=jax.ShapeDtypeStruct((C, H, W), jnp.float32),
        input_output_aliases={0: 0},
    )(partial, mask, x)
    return out


# speedup vs baseline: 1.1365x; 1.0933x over previous
import jax
import jax.numpy as jnp
from jax.experimental import pallas as pl
from jax.experimental.pallas import tpu as pltpu

C, H, W = 3, 4096, 4096
N_SITES = 1228
PAD_SITES = 1280
BH = 512


def _fused_kernel(sites_ref, x_ref, out_ref, mask_ref):
    c = pl.program_id(0)
    h = pl.program_id(1)

    @pl.when((c == 0) & (h == 0))
    def _build_mask():
        iota = jax.lax.broadcasted_iota(jnp.int32, (1, W), 1)
        acc = jnp.zeros((1, W), dtype=jnp.float32)
        for j in range(PAD_SITES // 128):
            row = sites_ref[j, :].reshape(128, 1)
            hit = jnp.any(row == iota, axis=0, keepdims=True)
            acc = jnp.maximum(acc, hit.astype(jnp.float32))
        mask_ref[...] = acc

    xb = x_ref[...]

    @pl.when(c < C - 1)
    def _copy():
        out_ref[...] = xb

    @pl.when(c == C - 1)
    def _flip():
        m = mask_ref[...].reshape(1, 1, W)
        out_ref[...] = jnp.where(m > 0.0, 1.0 - xb, xb)


@jax.jit
def kernel(x, mask_sites):
    sites = mask_sites.astype(jnp.int32)
    sites = jnp.pad(sites, (0, PAD_SITES - N_SITES), constant_values=jnp.int32(1 << 30))
    sites = sites.reshape(PAD_SITES // 128, 128)

    out = pl.pallas_call(
        _fused_kernel,
        grid=(C, H // BH),
        in_specs=[
            pl.BlockSpec((PAD_SITES // 128, 128), lambda c, h: (0, 0)),
            pl.BlockSpec((1, BH, W), lambda c, h: ((c + 1) % C, h, 0)),
        ],
        out_specs=pl.BlockSpec((1, BH, W), lambda c, h: ((c + 1) % C, h, 0)),
        out_shape=jax.ShapeDtypeStruct((C, H, W), jnp.float32),
        scratch_shapes=[pltpu.VMEM((1, W), jnp.float32)],
        compiler_params=pltpu.CompilerParams(vmem_limit_bytes=100 * 1024 * 1024),
    )(sites, x)
    return out


# speedup vs baseline: 1.1369x; 1.0003x over previous
import jax
import jax.numpy as jnp
from jax.experimental import pallas as pl
from jax.experimental.pallas import tpu as pltpu

BH = 512


def _make_fused_kernel(C, W, n_chunks):
    def _fused_kernel(sites_ref, x_ref, out_ref, mask_ref):
        c = pl.program_id(0)
        h = pl.program_id(1)

        @pl.when((c == 0) & (h == 0))
        def _build_mask():
            iota = jax.lax.broadcasted_iota(jnp.int32, (1, W), 1)
            acc = jnp.zeros((1, W), dtype=jnp.float32)
            for j in range(n_chunks):
                row = sites_ref[j, :].reshape(128, 1)
                hit = jnp.any(row == iota, axis=0, keepdims=True)
                acc = jnp.maximum(acc, hit.astype(jnp.float32))
            mask_ref[...] = acc

        xb = x_ref[...]

        @pl.when(c < C - 1)
        def _copy():
            out_ref[...] = xb

        @pl.when(c == C - 1)
        def _flip():
            m = mask_ref[...].reshape(1, 1, W)
            out_ref[...] = jnp.where(m > 0.0, 1.0 - xb, xb)

    return _fused_kernel


@jax.jit
def kernel(x, mask_sites):
    C, H, W = x.shape
    n_sites = mask_sites.shape[0]
    pad_sites = -(-n_sites // 128) * 128
    sites = mask_sites.astype(jnp.int32)
    sites = jnp.pad(sites, (0, pad_sites - n_sites), constant_values=jnp.int32(1 << 30))
    sites = sites.reshape(pad_sites // 128, 128)

    out = pl.pallas_call(
        _make_fused_kernel(C, W, pad_sites // 128),
        grid=(C, H // BH),
        in_specs=[
            pl.BlockSpec((pad_sites // 128, 128), lambda c, h: (0, 0)),
            pl.BlockSpec((1, BH, W), lambda c, h: ((c + 1) % C, h, 0)),
        ],
        out_specs=pl.BlockSpec((1, BH, W), lambda c, h: ((c + 1) % C, h, 0)),
        out_shape=jax.ShapeDtypeStruct((C, H, W), jnp.float32),
        scratch_shapes=[pltpu.VMEM((1, W), jnp.float32)],
    )(sites, x)
    return out
